# baseline (speedup 1.0000x reference)
import functools

import jax
import jax.numpy as jnp
from jax import lax
from jax.experimental import pallas as pl
from jax.experimental.pallas import tpu as pltpu
from jax.experimental.pallas import tpu_sc as plsc

_TILE_R = 16


def _make_sc_add(B, L, D):
    info = plsc.get_sparse_core_info()
    nw = info.num_cores * info.num_subcores
    rows_per_w = L // nw
    n_tiles = rows_per_w // _TILE_R
    mesh = plsc.VectorSubcoreMesh(core_axis_name="c", subcore_axis_name="s")

    @functools.partial(
        pl.kernel,
        mesh=mesh,
        out_type=jax.ShapeDtypeStruct((B, L, D), jnp.float32),
        scratch_types=[
            pltpu.VMEM((_TILE_R, D), jnp.float32),
            pltpu.VMEM((_TILE_R, D), jnp.float32),
        ],
    )
    def sc_add(x_hbm, t_hbm, out_hbm, bufx, buft):
        wid = lax.axis_index("s") * info.num_cores + lax.axis_index("c")
        base = wid * rows_per_w

        def tile_body(t, _):
            r0 = base + t * _TILE_R
            pltpu.sync_copy(t_hbm.at[pl.ds(r0, _TILE_R), :], buft)

            def batch_body(b, _):
                pltpu.sync_copy(x_hbm.at[b, pl.ds(r0, _TILE_R), :], bufx)

                def row_body(r, _):
                    def col_body(c, _):
                        c0 = c * 16
                        bufx[r, pl.ds(c0, 16)] = (
                            bufx[r, pl.ds(c0, 16)] + buft[r, pl.ds(c0, 16)]
                        )
                        return 0

                    return lax.fori_loop(0, D // 16, col_body, 0)

                lax.fori_loop(0, _TILE_R, row_body, 0)
                pltpu.sync_copy(bufx, out_hbm.at[b, pl.ds(r0, _TILE_R), :])
                return 0

            return lax.fori_loop(0, B, batch_body, 0)

        lax.fori_loop(0, n_tiles, tile_body, 0)

    return sc_add


def kernel(x, pos_table):
    B, L, D = x.shape
    return _make_sc_add(B, L, D)(x, pos_table[:L])


# speedup vs baseline: 1.5465x; 1.5465x over previous
import functools

import jax
import jax.numpy as jnp
from jax import lax
from jax.experimental import pallas as pl
from jax.experimental.pallas import tpu as pltpu
from jax.experimental.pallas import tpu_sc as plsc

_L_BLOCK = 512
_TILE_R = 16


def _tc_body(x_ref, t_ref, o_ref):
    o_ref[...] = x_ref[...] + t_ref[...][None, :, :]


def _tc_add(x, table):
    B, L, D = x.shape
    lb = min(_L_BLOCK, L)
    return pl.pallas_call(
        _tc_body,
        grid=(L // lb,),
        in_specs=[
            pl.BlockSpec((B, lb, D), lambda i: (0, i, 0)),
            pl.BlockSpec((lb, D), lambda i: (i, 0)),
        ],
        out_specs=pl.BlockSpec((B, lb, D), lambda i: (0, i, 0)),
        out_shape=jax.ShapeDtypeStruct((B, L, D), x.dtype),
    )(x, table)


def _make_sc_add(L, D):
    info = plsc.get_sparse_core_info()
    nw = info.num_cores * info.num_subcores
    rows_per_w = L // nw
    n_tiles = rows_per_w // _TILE_R
    mesh = plsc.VectorSubcoreMesh(core_axis_name="c", subcore_axis_name="s")

    @functools.partial(
        pl.kernel,
        mesh=mesh,
        out_type=jax.ShapeDtypeStruct((L, D), jnp.float32),
        scratch_types=[
            pltpu.VMEM((_TILE_R, D), jnp.float32),
            pltpu.VMEM((_TILE_R, D), jnp.float32),
        ],
    )
    def sc_add(x_hbm, t_hbm, out_hbm, bufx, buft):
        wid = lax.axis_index("s") * info.num_cores + lax.axis_index("c")
        base = wid * rows_per_w

        def tile_body(t, _):
            r0 = base + t * _TILE_R
            pltpu.sync_copy(t_hbm.at[pl.ds(r0, _TILE_R), :], buft)
            pltpu.sync_copy(x_hbm.at[pl.ds(r0, _TILE_R), :], bufx)

            def row_body(r, _):
                def col_body(c, _):
                    c0 = c * 16
                    bufx[r, pl.ds(c0, 16)] = (
                        bufx[r, pl.ds(c0, 16)] + buft[r, pl.ds(c0, 16)]
                    )
                    return 0

                return lax.fori_loop(0, D // 16, col_body, 0)

            lax.fori_loop(0, _TILE_R, row_body, 0)
            pltpu.sync_copy(bufx, out_hbm.at[pl.ds(r0, _TILE_R), :])
            return 0

        lax.fori_loop(0, n_tiles, tile_body, 0)

    return sc_add


def kernel(x, pos_table):
    B, L, D = x.shape
    table = pos_table[:L]
    sc_out = _make_sc_add(L, D)(x[B - 1], table)
    tc_out = _tc_add(x[: B - 1], table)
    return jnp.concatenate([tc_out, sc_out[None]], axis=0)


# speedup vs baseline: 5.3218x; 3.4411x over previous
import jax
import jax.numpy as jnp
from jax.experimental import pallas as pl

_L_BLOCK = 1024
_B_BLOCK = 2


def _add_body(x_ref, t_ref, o_ref):
    o_ref[...] = x_ref[...] + t_ref[...][None, :, :]


def kernel(x, pos_table):
    B, L, D = x.shape
    lb = min(_L_BLOCK, L)
    bb = min(_B_BLOCK, B)
    return pl.pallas_call(
        _add_body,
        grid=(L // lb, B // bb),
        in_specs=[
            pl.BlockSpec((bb, lb, D), lambda i, j: (j, i, 0)),
            pl.BlockSpec((lb, D), lambda i, j: (i, 0)),
        ],
        out_specs=pl.BlockSpec((bb, lb, D), lambda i, j: (j, i, 0)),
        out_shape=jax.ShapeDtypeStruct((B, L, D), x.dtype),
    )(x, pos_table[:L])
